# baseline (speedup 1.0000x reference)
import functools

import jax
import jax.numpy as jnp
from jax import lax
from jax.experimental import pallas as pl
from jax.experimental.pallas import tpu as pltpu
from jax.experimental.pallas import tpu_sc as plsc

HIDDEN_DIM = 512
BATCH = 16384
LANES = 16

_info = plsc.get_sparse_core_info()
NC, NS = _info.num_cores, _info.num_subcores
NW = NC * NS
B_PER_W = BATCH // NW

QROWS = 2
NQPAT = 2 ** QROWS
NGRP = B_PER_W // LANES
N_Q = B_PER_W // QROWS

NCOL = 4
CW = HIDDEN_DIM // NCOL
JH = CW // LANES


def _perm(x, idx):
    return lax.gather(
        x, idx.reshape(LANES, 1),
        lax.GatherDimensionNumbers(
            offset_dims=(), collapsed_slice_dims=(0,), start_index_map=(0,)),
        (1,), mode=lax.GatherScatterMode.PROMISE_IN_BOUNDS)


def _mesh_kernel():
    mesh = plsc.VectorSubcoreMesh(core_axis_name="c", subcore_axis_name="s")

    @functools.partial(
        pl.kernel,
        mesh=mesh,
        out_type=jax.ShapeDtypeStruct((BATCH, HIDDEN_DIM), jnp.float32),
        scratch_types=[
            pltpu.VMEM((B_PER_W,), jnp.int32),
            pltpu.VMEM((2, HIDDEN_DIM), jnp.float32),
            pltpu.VMEM((NQPAT, QROWS, HIDDEN_DIM), jnp.float32),
            pltpu.SemaphoreType.DMA,
            pltpu.SemaphoreType.DMA,
            pltpu.SemaphoreType.DMA,
        ],
    )
    def body(table_hbm, idx_hbm, out_hbm, idx_v, tab_v, pairs,
             sem, semi, semt):
        wid = lax.axis_index("s") * NC + lax.axis_index("c")
        base = wid * B_PER_W
        cp_idx = pltpu.async_copy(idx_hbm.at[wid], idx_v, semi)
        cp_tab = pltpu.async_copy(table_hbm, tab_v, semt)

        lane = lax.iota(jnp.int32, LANES)
        wvec = 2 >> jnp.bitwise_and(lane, 1)

        cp_tab.wait()
        for h in range(NCOL):
            c0 = h * CW
            w0 = [tab_v[0, pl.ds(c0 + j * LANES, LANES)] for j in range(JH)]
            w1 = [tab_v[1, pl.ds(c0 + j * LANES, LANES)] for j in range(JH)]
            for q in range(NQPAT):
                for rr in range(QROWS):
                    src = w1 if (q >> (QROWS - 1 - rr)) & 1 else w0
                    for j in range(JH):
                        pairs[q, rr, pl.ds(c0 + j * LANES, LANES)] = src[j]

        cp_idx.wait()

        def issue_body(t, _):
            v = idx_v[pl.ds(t * LANES, LANES)]
            s = v * wvec
            s = s + _perm(s, jnp.bitwise_xor(lane, 1))
            row0 = base + t * LANES
            for i in range(LANES // QROWS):
                pltpu.async_copy(
                    pairs.at[s[QROWS * i]],
                    out_hbm.at[pl.ds(row0 + i * QROWS, QROWS)], sem)
            return 0

        lax.fori_loop(0, NGRP, issue_body, 0)

        def drain_body(t, _):
            pltpu.make_async_copy(
                pairs.at[0], out_hbm.at[pl.ds(base, QROWS)], sem).wait()
            return 0

        lax.fori_loop(0, N_Q, drain_body, 0)

    return body


_sc_lookup = _mesh_kernel()


@jax.jit
def kernel(domain_ids, embed_weight):
    ids = domain_ids.astype(jnp.int32).reshape(NW, B_PER_W)
    return _sc_lookup(embed_weight, ids)
